# Initial kernel scaffold; baseline (speedup 1.0000x reference)
import jax
import jax.numpy as jnp
from jax.experimental import pallas as pl


def kernel(inputs, is_training, embeddings):
    raise NotImplementedError("write your pallas kernel here")


# speedup vs baseline: 2.3910x; 2.3910x over previous
import jax
import jax.numpy as jnp
from jax.experimental import pallas as pl
from jax.experimental.pallas import tpu as pltpu

_D = 64
_K = 1024
_N = 16 * 32 * 32
_TILE_M = 512
_NT = _N // _TILE_M
_COMMIT = 0.25


def _vq_body(x_ref, embt_ref, emb_ref,
             dist_ref, enc_ref, q_ref, idx_ref, loss_ref, perp_ref,
             cnt_ref, sse_ref):
    i = pl.program_id(0)
    x = x_ref[...]
    embt = embt_ref[...]
    x2 = jnp.sum(x * x, axis=1, keepdims=True)
    e2 = jnp.sum(embt * embt, axis=0, keepdims=True)
    mm = jnp.matmul(x, embt, preferred_element_type=jnp.float32)
    dist = (x2 - 2.0 * mm) + e2
    dist_ref[...] = dist

    idx = jnp.argmin(dist, axis=1).astype(jnp.int32)
    idx_ref[...] = idx.reshape(1, 1, _TILE_M)
    iota = jax.lax.broadcasted_iota(jnp.int32, (_TILE_M, _K), 1)
    enc = (iota == idx[:, None]).astype(jnp.float32)
    enc_ref[...] = enc
    q_ref[...] = jnp.matmul(enc, emb_ref[...],
                            preferred_element_type=jnp.float32)

    mind = jnp.min(dist, axis=1, keepdims=True)
    ssum = jnp.sum(mind, keepdims=True)
    cnt = jnp.sum(enc, axis=0, keepdims=True)

    @pl.when(i == 0)
    def _():
        cnt_ref[...] = cnt
        sse_ref[...] = ssum

    @pl.when(i > 0)
    def _():
        cnt_ref[...] += cnt
        sse_ref[...] += ssum

    @pl.when(i == _NT - 1)
    def _():
        loss_ref[...] = (1.0 + _COMMIT) * (sse_ref[...] / (_N * _D))
        avg = cnt_ref[...] / _N
        ent = jnp.sum(avg * jnp.log(avg + 1e-10), keepdims=True)
        perp_ref[...] = jnp.exp(-ent)


def kernel(inputs, is_training, embeddings):
    x = jnp.transpose(inputs, (0, 3, 2, 1))
    flat = x.reshape(_N, _D)
    embt = embeddings.T

    dist, enc, q, idx3, loss, perp = pl.pallas_call(
        _vq_body,
        grid=(_NT,),
        in_specs=[
            pl.BlockSpec((_TILE_M, _D), lambda i: (i, 0)),
            pl.BlockSpec((_D, _K), lambda i: (0, 0)),
            pl.BlockSpec((_K, _D), lambda i: (0, 0)),
        ],
        out_specs=[
            pl.BlockSpec((_TILE_M, _K), lambda i: (i, 0)),
            pl.BlockSpec((_TILE_M, _K), lambda i: (i, 0)),
            pl.BlockSpec((_TILE_M, _D), lambda i: (i, 0)),
            pl.BlockSpec((1, 1, _TILE_M), lambda i: (i, 0, 0)),
            pl.BlockSpec((1, 1), lambda i: (0, 0)),
            pl.BlockSpec((1, 1), lambda i: (0, 0)),
        ],
        out_shape=[
            jax.ShapeDtypeStruct((_N, _K), jnp.float32),
            jax.ShapeDtypeStruct((_N, _K), jnp.float32),
            jax.ShapeDtypeStruct((_N, _D), jnp.float32),
            jax.ShapeDtypeStruct((_NT, 1, _TILE_M), jnp.int32),
            jax.ShapeDtypeStruct((1, 1), jnp.float32),
            jax.ShapeDtypeStruct((1, 1), jnp.float32),
        ],
        scratch_shapes=[
            pltpu.VMEM((1, _K), jnp.float32),
            pltpu.VMEM((1, 1), jnp.float32),
        ],
    )(flat, embt, embeddings)

    enc_idx = idx3.reshape(16, 32, 32)
    quantize = jnp.transpose(q.reshape(16, 32, 32, _D), (0, 3, 2, 1))
    return (quantize, loss[0, 0], perp[0, 0], enc, enc_idx, dist)


# speedup vs baseline: 2.5527x; 1.0676x over previous
import jax
import jax.numpy as jnp
from jax.experimental import pallas as pl
from jax.experimental.pallas import tpu as pltpu

_D = 64
_K = 1024
_N = 16 * 32 * 32
_TILE_M = 1024
_NT = _N // _TILE_M
_COMMIT = 0.25


def _vq_body(x_ref, embt_ref, emb_ref,
             dist_ref, enc_ref, q_ref, idx_ref, loss_ref, perp_ref,
             e2_ref, cnt_ref, sse_ref):
    i = pl.program_id(0)
    embt = embt_ref[...]

    @pl.when(i == 0)
    def _():
        e2_ref[...] = jnp.sum(embt * embt, axis=0, keepdims=True)

    x = x_ref[...]
    x2 = jnp.sum(x * x, axis=1, keepdims=True)
    mm = jnp.matmul(x, embt, preferred_element_type=jnp.float32)
    dist = (x2 - 2.0 * mm) + e2_ref[...]
    dist_ref[...] = dist

    mind = jnp.min(dist, axis=1, keepdims=True)
    iota = jax.lax.broadcasted_iota(jnp.int32, (_TILE_M, _K), 1)
    idx = jnp.min(jnp.where(dist == mind, iota, _K), axis=1)
    idx_ref[...] = idx.reshape(1, 1, _TILE_M)
    enc = (iota == idx[:, None]).astype(jnp.float32)
    enc_ref[...] = enc
    q_ref[...] = jnp.matmul(enc, emb_ref[...],
                            preferred_element_type=jnp.float32)

    ssum = jnp.sum(mind, keepdims=True)
    cnt = jnp.sum(enc, axis=0, keepdims=True)

    @pl.when(i == 0)
    def _():
        cnt_ref[...] = cnt
        sse_ref[...] = ssum

    @pl.when(i > 0)
    def _():
        cnt_ref[...] += cnt
        sse_ref[...] += ssum

    @pl.when(i == _NT - 1)
    def _():
        loss_ref[...] = (1.0 + _COMMIT) * (sse_ref[...] / (_N * _D))
        avg = cnt_ref[...] / _N
        ent = jnp.sum(avg * jnp.log(avg + 1e-10), keepdims=True)
        perp_ref[...] = jnp.exp(-ent)


def kernel(inputs, is_training, embeddings):
    x = jnp.transpose(inputs, (0, 3, 2, 1))
    flat = x.reshape(_N, _D)
    embt = embeddings.T

    dist, enc, q, idx3, loss, perp = pl.pallas_call(
        _vq_body,
        grid=(_NT,),
        in_specs=[
            pl.BlockSpec((_TILE_M, _D), lambda i: (i, 0)),
            pl.BlockSpec((_D, _K), lambda i: (0, 0)),
            pl.BlockSpec((_K, _D), lambda i: (0, 0)),
        ],
        out_specs=[
            pl.BlockSpec((_TILE_M, _K), lambda i: (i, 0)),
            pl.BlockSpec((_TILE_M, _K), lambda i: (i, 0)),
            pl.BlockSpec((_TILE_M, _D), lambda i: (i, 0)),
            pl.BlockSpec((1, 1, _TILE_M), lambda i: (i, 0, 0)),
            pl.BlockSpec((1, 1), lambda i: (0, 0)),
            pl.BlockSpec((1, 1), lambda i: (0, 0)),
        ],
        out_shape=[
            jax.ShapeDtypeStruct((_N, _K), jnp.float32),
            jax.ShapeDtypeStruct((_N, _K), jnp.float32),
            jax.ShapeDtypeStruct((_N, _D), jnp.float32),
            jax.ShapeDtypeStruct((_NT, 1, _TILE_M), jnp.int32),
            jax.ShapeDtypeStruct((1, 1), jnp.float32),
            jax.ShapeDtypeStruct((1, 1), jnp.float32),
        ],
        scratch_shapes=[
            pltpu.VMEM((1, _K), jnp.float32),
            pltpu.VMEM((1, _K), jnp.float32),
            pltpu.VMEM((1, 1), jnp.float32),
        ],
    )(flat, embt, embeddings)

    enc_idx = idx3.reshape(16, 32, 32)
    quantize = jnp.transpose(q.reshape(16, 32, 32, _D), (0, 3, 2, 1))
    return (quantize, loss[0, 0], perp[0, 0], enc, enc_idx, dist)
